# baseline (speedup 1.0000x reference)
import functools

import jax
import jax.numpy as jnp
from jax import lax
from jax.experimental import pallas as pl
from jax.experimental.pallas import tpu as pltpu
from jax.experimental.pallas import tpu_sc as plsc

N_V = 10000
N_F = 10000
DIM = 128
LANES = 16

CHUNK = 40
NCHUNKS = N_F // CHUNK
MAXT = 8
IPAD = 32 * MAXT * CHUNK


def _face_features_sc(primal, dual, i0, i1, i2):
    info = plsc.get_sparse_core_info()
    nc, ns = info.num_cores, info.num_subcores
    nw = nc * ns
    mesh = plsc.VectorSubcoreMesh(core_axis_name="c", subcore_axis_name="s")
    full = NCHUNKS - nw * (MAXT - 1)

    rows = pltpu.VMEM((CHUNK, DIM), jnp.float32)

    @functools.partial(
        pl.kernel,
        mesh=mesh,
        out_type=jax.ShapeDtypeStruct((N_F, DIM), jnp.float32),
        scratch_types=[
            pltpu.VMEM((MAXT * CHUNK,), jnp.int32),
            pltpu.VMEM((MAXT * CHUNK,), jnp.int32),
            pltpu.VMEM((MAXT * CHUNK,), jnp.int32),
            rows, rows, rows, rows, rows,
            rows, rows, rows, rows, rows,
            pltpu.SemaphoreType.DMA,
            pltpu.SemaphoreType.DMA,
            pltpu.SemaphoreType.DMA,
            pltpu.SemaphoreType.DMA,
        ],
    )
    def sc_kernel(primal_hbm, dual_hbm, i0_hbm, i1_hbm, i2_hbm, f_hbm,
                  iva, ivb, ivc,
                  p0a, p1a, p2a, dva, fva,
                  p0b, p1b, p2b, dvb, fvb,
                  sema, semb, wsema, wsemb):
        wid = lax.axis_index("s") * nc + lax.axis_index("c")
        start = wid * MAXT - jnp.maximum(wid - full, 0)
        count = jnp.where(wid < full, MAXT, MAXT - 1)
        base0 = start * CHUNK
        slots = [
            (p0a, p1a, p2a, dva, fva, sema, wsema),
            (p0b, p1b, p2b, dvb, fvb, semb, wsemb),
        ]

        pltpu.sync_copy(i0_hbm.at[pl.ds(base0, MAXT * CHUNK)], iva)
        pltpu.sync_copy(i1_hbm.at[pl.ds(base0, MAXT * CHUNK)], ivb)
        pltpu.sync_copy(i2_hbm.at[pl.ds(base0, MAXT * CHUNK)], ivc)

        def fire(t):
            p0, p1, p2, dv, fv, sem, wsem = slots[t % 2]
            isl = pl.ds(t * CHUNK, CHUNK)

            @pl.when(t < count)
            def _():
                pltpu.async_copy(primal_hbm.at[iva.at[isl]], p0, sem)
                pltpu.async_copy(primal_hbm.at[ivb.at[isl]], p1, sem)
                pltpu.async_copy(primal_hbm.at[ivc.at[isl]], p2, sem)
                pltpu.async_copy(
                    dual_hbm.at[pl.ds(base0 + t * CHUNK, CHUNK)], dv, sem)

        def drain(t):
            p0, p1, p2, dv, fv, sem, wsem = slots[t % 2]
            isl = pl.ds(t * CHUNK, CHUNK)

            @pl.when(t < count)
            def _():
                pltpu.make_async_copy(primal_hbm.at[iva.at[isl]], p0, sem).wait()
                pltpu.make_async_copy(primal_hbm.at[ivb.at[isl]], p1, sem).wait()
                pltpu.make_async_copy(primal_hbm.at[ivc.at[isl]], p2, sem).wait()
                pltpu.make_async_copy(
                    dual_hbm.at[pl.ds(base0 + t * CHUNK, CHUNK)], dv, sem).wait()

        def drain_write(t):
            p0, p1, p2, dv, fv, sem, wsem = slots[t % 2]

            @pl.when(t < count)
            def _():
                pltpu.make_async_copy(
                    fv, f_hbm.at[pl.ds(base0 + t * CHUNK, CHUNK)], wsem).wait()

        def compute_and_store(t):
            p0, p1, p2, dv, fv, sem, wsem = slots[t % 2]

            @pl.when(t < count)
            def _():
                def body(r, carry):
                    for d in range(DIM // LANES):
                        sl = pl.ds(d * LANES, LANES)
                        dd = dv[r, sl]
                        acc = (jnp.abs(p0[r, sl] - dd)
                               + jnp.abs(p1[r, sl] - dd)
                               + jnp.abs(p2[r, sl] - dd))
                        fv[r, sl] = acc * jnp.float32(1.0 / 3.0)
                    return carry

                lax.fori_loop(0, CHUNK, body, 0)
                pltpu.async_copy(
                    fv, f_hbm.at[pl.ds(base0 + t * CHUNK, CHUNK)], wsem)

        fire(0)
        for t in range(MAXT):
            if t + 1 < MAXT:
                fire(t + 1)
            drain(t)
            if t >= 2:
                drain_write(t - 2)
            compute_and_store(t)
        drain_write(MAXT - 2)
        drain_write(MAXT - 1)

    return sc_kernel(primal, dual, i0, i1, i2)


BI = 400
NBI = N_V // BI
BH = BI // 2


def _tc_fused_body(a_top_ref, a_bot_ref, f_all_ref, f_blk_ref,
                   primal_ref, dual_ref,
                   wp1_ref, wp2_ref, wd1_ref, wd2_ref, bp_ref, bd_ref,
                   outp_ref, outd_ref):
    f_all = f_all_ref[...]
    mapped_top = jnp.dot(a_top_ref[...], f_all,
                         preferred_element_type=jnp.float32)
    outp_ref[:BH, :] = jnp.maximum(
        jnp.dot(primal_ref[:BH, :], wp1_ref[...],
                preferred_element_type=jnp.float32)
        + jnp.dot(mapped_top, wp2_ref[...], preferred_element_type=jnp.float32)
        + bp_ref[...], 0.0)
    mapped_bot = jnp.dot(a_bot_ref[...], f_all,
                         preferred_element_type=jnp.float32)
    outp_ref[BH:, :] = jnp.maximum(
        jnp.dot(primal_ref[BH:, :], wp1_ref[...],
                preferred_element_type=jnp.float32)
        + jnp.dot(mapped_bot, wp2_ref[...], preferred_element_type=jnp.float32)
        + bp_ref[...], 0.0)
    outd_ref[...] = jnp.maximum(
        jnp.dot(dual_ref[...], wd1_ref[...],
                preferred_element_type=jnp.float32)
        + jnp.dot(f_blk_ref[...], wd2_ref[...],
                  preferred_element_type=jnp.float32)
        + bd_ref[...], 0.0)


def _tc_fused(A, f, primal, dual, wp1, wp2, wd1, wd2, bp, bd):
    out_shape = (
        jax.ShapeDtypeStruct((N_V, DIM), jnp.float32),
        jax.ShapeDtypeStruct((N_F, DIM), jnp.float32),
    )
    dimdim = pl.BlockSpec((DIM, DIM), lambda i: (0, 0))
    rowblk = pl.BlockSpec((BI, DIM), lambda i: (i, 0))
    bias = pl.BlockSpec((1, DIM), lambda i: (0, 0))
    nh = N_V // BH
    return pl.pallas_call(
        _tc_fused_body,
        grid=(NBI,),
        in_specs=[
            pl.BlockSpec((BH, N_F), lambda i: (2 * i, 0)),
            pl.BlockSpec((BH, N_F), lambda i: (2 * i + 1, 0)),
            pl.BlockSpec((N_F, DIM), lambda i: (0, 0)),
            rowblk,
            rowblk,
            rowblk,
            dimdim, dimdim, dimdim, dimdim,
            bias, bias,
        ],
        out_specs=(rowblk, rowblk),
        out_shape=out_shape,
        compiler_params=pltpu.CompilerParams(
            dimension_semantics=("arbitrary",),
        ),
    )(A, A, f, f, primal, dual, wp1, wp2, wd1, wd2, bp, bd)


def kernel(primal, dual, A, faces, W_primal, b_primal, W_dual, b_dual):
    faces = faces.astype(jnp.int32)
    pad = IPAD - N_F
    i0 = jnp.pad(faces[:, 0], (0, pad))
    i1 = jnp.pad(faces[:, 1], (0, pad))
    i2 = jnp.pad(faces[:, 2], (0, pad))
    f = _face_features_sc(primal, dual, i0, i1, i2)
    wp1 = W_primal[:, :DIM].T
    wp2 = W_primal[:, DIM:].T
    wd1 = W_dual[:, :DIM].T
    wd2 = W_dual[:, DIM:].T
    bp = b_primal.reshape(1, DIM)
    bd = b_dual.reshape(1, DIM)
    out_primal, out_dual = _tc_fused(A, f, primal, dual,
                                     wp1, wp2, wd1, wd2, bp, bd)
    return (out_primal, out_dual)


# speedup vs baseline: 1.0385x; 1.0385x over previous
import functools

import jax
import jax.numpy as jnp
from jax import lax
from jax.experimental import pallas as pl
from jax.experimental.pallas import tpu as pltpu
from jax.experimental.pallas import tpu_sc as plsc

N_V = 10000
N_F = 10000
DIM = 128
LANES = 16

CHUNK = 80
NCHUNKS = N_F // CHUNK
MAXT = 4
IPAD = 32 * MAXT * CHUNK


def _face_features_sc(primal, dual, i0, i1, i2):
    info = plsc.get_sparse_core_info()
    nc, ns = info.num_cores, info.num_subcores
    nw = nc * ns
    mesh = plsc.VectorSubcoreMesh(core_axis_name="c", subcore_axis_name="s")
    full = NCHUNKS - nw * (MAXT - 1)

    rows = pltpu.VMEM((CHUNK, DIM), jnp.float32)

    @functools.partial(
        pl.kernel,
        mesh=mesh,
        out_type=jax.ShapeDtypeStruct((N_F, DIM), jnp.float32),
        scratch_types=[
            pltpu.VMEM((MAXT * CHUNK,), jnp.int32),
            pltpu.VMEM((MAXT * CHUNK,), jnp.int32),
            pltpu.VMEM((MAXT * CHUNK,), jnp.int32),
            rows, rows, rows, rows, rows,
            rows, rows, rows, rows, rows,
            pltpu.SemaphoreType.DMA,
            pltpu.SemaphoreType.DMA,
            pltpu.SemaphoreType.DMA,
            pltpu.SemaphoreType.DMA,
        ],
    )
    def sc_kernel(primal_hbm, dual_hbm, i0_hbm, i1_hbm, i2_hbm, f_hbm,
                  iva, ivb, ivc,
                  p0a, p1a, p2a, dva, fva,
                  p0b, p1b, p2b, dvb, fvb,
                  sema, semb, wsema, wsemb):
        wid = lax.axis_index("s") * nc + lax.axis_index("c")
        start = wid * MAXT - jnp.maximum(wid - full, 0)
        count = jnp.where(wid < full, MAXT, MAXT - 1)
        base0 = start * CHUNK
        slots = [
            (p0a, p1a, p2a, dva, fva, sema, wsema),
            (p0b, p1b, p2b, dvb, fvb, semb, wsemb),
        ]

        pltpu.sync_copy(i0_hbm.at[pl.ds(base0, MAXT * CHUNK)], iva)
        pltpu.sync_copy(i1_hbm.at[pl.ds(base0, MAXT * CHUNK)], ivb)
        pltpu.sync_copy(i2_hbm.at[pl.ds(base0, MAXT * CHUNK)], ivc)

        def fire(t):
            p0, p1, p2, dv, fv, sem, wsem = slots[t % 2]
            isl = pl.ds(t * CHUNK, CHUNK)

            @pl.when(t < count)
            def _():
                pltpu.async_copy(primal_hbm.at[iva.at[isl]], p0, sem)
                pltpu.async_copy(primal_hbm.at[ivb.at[isl]], p1, sem)
                pltpu.async_copy(primal_hbm.at[ivc.at[isl]], p2, sem)
                pltpu.async_copy(
                    dual_hbm.at[pl.ds(base0 + t * CHUNK, CHUNK)], dv, sem)

        def drain(t):
            p0, p1, p2, dv, fv, sem, wsem = slots[t % 2]
            isl = pl.ds(t * CHUNK, CHUNK)

            @pl.when(t < count)
            def _():
                pltpu.make_async_copy(primal_hbm.at[iva.at[isl]], p0, sem).wait()
                pltpu.make_async_copy(primal_hbm.at[ivb.at[isl]], p1, sem).wait()
                pltpu.make_async_copy(primal_hbm.at[ivc.at[isl]], p2, sem).wait()
                pltpu.make_async_copy(
                    dual_hbm.at[pl.ds(base0 + t * CHUNK, CHUNK)], dv, sem).wait()

        def drain_write(t):
            p0, p1, p2, dv, fv, sem, wsem = slots[t % 2]

            @pl.when(t < count)
            def _():
                pltpu.make_async_copy(
                    fv, f_hbm.at[pl.ds(base0 + t * CHUNK, CHUNK)], wsem).wait()

        def compute_and_store(t):
            p0, p1, p2, dv, fv, sem, wsem = slots[t % 2]

            @pl.when(t < count)
            def _():
                def body(r, carry):
                    for d in range(DIM // LANES):
                        sl = pl.ds(d * LANES, LANES)
                        dd = dv[r, sl]
                        acc = (jnp.abs(p0[r, sl] - dd)
                               + jnp.abs(p1[r, sl] - dd)
                               + jnp.abs(p2[r, sl] - dd))
                        fv[r, sl] = acc * jnp.float32(1.0 / 3.0)
                    return carry

                lax.fori_loop(0, CHUNK, body, 0)
                pltpu.async_copy(
                    fv, f_hbm.at[pl.ds(base0 + t * CHUNK, CHUNK)], wsem)

        fire(0)
        for t in range(MAXT):
            if t + 1 < MAXT:
                fire(t + 1)
            drain(t)
            if t >= 2:
                drain_write(t - 2)
            compute_and_store(t)
        drain_write(MAXT - 2)
        drain_write(MAXT - 1)

    return sc_kernel(primal, dual, i0, i1, i2)


BI = 400
NBI = N_V // BI


def _tc_fused_body(a_ref, f_all_ref, f_blk_ref, primal_ref, dual_ref,
                   wp1_ref, wp2_ref, wd1_ref, wd2_ref, bp_ref, bd_ref,
                   outp_ref, outd_ref):
    mapped = jnp.dot(a_ref[...], f_all_ref[...],
                     preferred_element_type=jnp.float32)
    outp_ref[...] = jnp.maximum(
        jnp.dot(primal_ref[...], wp1_ref[...],
                preferred_element_type=jnp.float32)
        + jnp.dot(mapped, wp2_ref[...], preferred_element_type=jnp.float32)
        + bp_ref[...], 0.0)
    outd_ref[...] = jnp.maximum(
        jnp.dot(dual_ref[...], wd1_ref[...],
                preferred_element_type=jnp.float32)
        + jnp.dot(f_blk_ref[...], wd2_ref[...],
                  preferred_element_type=jnp.float32)
        + bd_ref[...], 0.0)


def _tc_fused(A, f, primal, dual, wp1, wp2, wd1, wd2, bp, bd):
    out_shape = (
        jax.ShapeDtypeStruct((N_V, DIM), jnp.float32),
        jax.ShapeDtypeStruct((N_F, DIM), jnp.float32),
    )
    dimdim = pl.BlockSpec((DIM, DIM), lambda i: (0, 0))
    rowblk = pl.BlockSpec((BI, DIM), lambda i: (i, 0))
    bias = pl.BlockSpec((1, DIM), lambda i: (0, 0))
    return pl.pallas_call(
        _tc_fused_body,
        grid=(NBI,),
        in_specs=[
            pl.BlockSpec((BI, N_F), lambda i: (i, 0)),
            pl.BlockSpec((N_F, DIM), lambda i: (0, 0)),
            rowblk,
            rowblk,
            rowblk,
            dimdim, dimdim, dimdim, dimdim,
            bias, bias,
        ],
        out_specs=(rowblk, rowblk),
        out_shape=out_shape,
        compiler_params=pltpu.CompilerParams(
            dimension_semantics=("arbitrary",),
        ),
    )(A, f, f, primal, dual, wp1, wp2, wd1, wd2, bp, bd)


def kernel(primal, dual, A, faces, W_primal, b_primal, W_dual, b_dual):
    faces = faces.astype(jnp.int32)
    pad = IPAD - N_F
    i0 = jnp.pad(faces[:, 0], (0, pad))
    i1 = jnp.pad(faces[:, 1], (0, pad))
    i2 = jnp.pad(faces[:, 2], (0, pad))
    f = _face_features_sc(primal, dual, i0, i1, i2)
    wp1 = W_primal[:, :DIM].T
    wp2 = W_primal[:, DIM:].T
    wd1 = W_dual[:, :DIM].T
    wd2 = W_dual[:, DIM:].T
    bp = b_primal.reshape(1, DIM)
    bd = b_dual.reshape(1, DIM)
    out_primal, out_dual = _tc_fused(A, f, primal, dual,
                                     wp1, wp2, wd1, wd2, bp, bd)
    return (out_primal, out_dual)


# speedup vs baseline: 1.0511x; 1.0121x over previous
import functools

import jax
import jax.numpy as jnp
from jax import lax
from jax.experimental import pallas as pl
from jax.experimental.pallas import tpu as pltpu
from jax.experimental.pallas import tpu_sc as plsc

N_V = 10000
N_F = 10000
DIM = 128
LANES = 16

CHUNK = 80
NCHUNKS = N_F // CHUNK
MAXT = 4
IPAD = 32 * MAXT * CHUNK


def _face_features_sc(primal, dual, i0, i1, i2):
    info = plsc.get_sparse_core_info()
    nc, ns = info.num_cores, info.num_subcores
    nw = nc * ns
    mesh = plsc.VectorSubcoreMesh(core_axis_name="c", subcore_axis_name="s")
    full = NCHUNKS - nw * (MAXT - 1)

    rows = pltpu.VMEM((CHUNK, DIM), jnp.float32)

    @functools.partial(
        pl.kernel,
        mesh=mesh,
        out_type=jax.ShapeDtypeStruct((N_F, DIM), jnp.float32),
        scratch_types=[
            pltpu.VMEM((MAXT * CHUNK,), jnp.int32),
            pltpu.VMEM((MAXT * CHUNK,), jnp.int32),
            pltpu.VMEM((MAXT * CHUNK,), jnp.int32),
            rows, rows, rows, rows, rows,
            rows, rows, rows, rows, rows,
            pltpu.SemaphoreType.DMA,
            pltpu.SemaphoreType.DMA,
            pltpu.SemaphoreType.DMA,
            pltpu.SemaphoreType.DMA,
        ],
    )
    def sc_kernel(primal_hbm, dual_hbm, i0_hbm, i1_hbm, i2_hbm, f_hbm,
                  iva, ivb, ivc,
                  p0a, p1a, p2a, dva, fva,
                  p0b, p1b, p2b, dvb, fvb,
                  sema, semb, wsema, wsemb):
        wid = lax.axis_index("s") * nc + lax.axis_index("c")
        start = wid * MAXT - jnp.maximum(wid - full, 0)
        count = jnp.where(wid < full, MAXT, MAXT - 1)
        base0 = start * CHUNK
        slots = [
            (p0a, p1a, p2a, dva, fva, sema, wsema),
            (p0b, p1b, p2b, dvb, fvb, semb, wsemb),
        ]

        pltpu.sync_copy(i0_hbm.at[pl.ds(base0, MAXT * CHUNK)], iva)
        pltpu.sync_copy(i1_hbm.at[pl.ds(base0, MAXT * CHUNK)], ivb)
        pltpu.sync_copy(i2_hbm.at[pl.ds(base0, MAXT * CHUNK)], ivc)

        def fire(t):
            p0, p1, p2, dv, fv, sem, wsem = slots[t % 2]
            isl = pl.ds(t * CHUNK, CHUNK)

            @pl.when(t < count)
            def _():
                pltpu.async_copy(primal_hbm.at[iva.at[isl]], p0, sem)
                pltpu.async_copy(primal_hbm.at[ivb.at[isl]], p1, sem)
                pltpu.async_copy(primal_hbm.at[ivc.at[isl]], p2, sem)
                pltpu.async_copy(
                    dual_hbm.at[pl.ds(base0 + t * CHUNK, CHUNK)], dv, sem)

        def drain(t):
            p0, p1, p2, dv, fv, sem, wsem = slots[t % 2]
            isl = pl.ds(t * CHUNK, CHUNK)

            @pl.when(t < count)
            def _():
                pltpu.make_async_copy(primal_hbm.at[iva.at[isl]], p0, sem).wait()
                pltpu.make_async_copy(primal_hbm.at[ivb.at[isl]], p1, sem).wait()
                pltpu.make_async_copy(primal_hbm.at[ivc.at[isl]], p2, sem).wait()
                pltpu.make_async_copy(
                    dual_hbm.at[pl.ds(base0 + t * CHUNK, CHUNK)], dv, sem).wait()

        def drain_write(t):
            p0, p1, p2, dv, fv, sem, wsem = slots[t % 2]

            @pl.when(t < count)
            def _():
                pltpu.make_async_copy(
                    fv, f_hbm.at[pl.ds(base0 + t * CHUNK, CHUNK)], wsem).wait()

        def compute_and_store(t):
            p0, p1, p2, dv, fv, sem, wsem = slots[t % 2]

            @pl.when(t < count)
            def _():
                def body(r, carry):
                    for d in range(DIM // LANES):
                        sl = pl.ds(d * LANES, LANES)
                        dd = dv[r, sl]
                        acc = (jnp.abs(p0[r, sl] - dd)
                               + jnp.abs(p1[r, sl] - dd)
                               + jnp.abs(p2[r, sl] - dd))
                        fv[r, sl] = acc * jnp.float32(1.0 / 3.0)
                    return carry

                lax.fori_loop(0, CHUNK, body, 0)
                pltpu.async_copy(
                    fv, f_hbm.at[pl.ds(base0 + t * CHUNK, CHUNK)], wsem)

        fire(0)
        for t in range(MAXT):
            if t + 1 < MAXT:
                fire(t + 1)
            drain(t)
            if t >= 2:
                drain_write(t - 2)
            compute_and_store(t)
        drain_write(MAXT - 2)
        drain_write(MAXT - 1)

    return sc_kernel(primal, dual, i0, i1, i2)


BI = 400
NBI = N_V // BI


def _tc_fused_body(a_ref, f_all_ref, primal_ref, dual_ref,
                   wp1_ref, wp2_ref, wd1_ref, wd2_ref, bp_ref, bd_ref,
                   outp_ref, outd_ref):
    i = pl.program_id(0)
    f_all = f_all_ref[...]
    mapped = jnp.dot(a_ref[...], f_all, preferred_element_type=jnp.float32)
    outp_ref[...] = jnp.maximum(
        jnp.dot(primal_ref[...], wp1_ref[...],
                preferred_element_type=jnp.float32)
        + jnp.dot(mapped, wp2_ref[...], preferred_element_type=jnp.float32)
        + bp_ref[...], 0.0)
    f_blk = f_all_ref[pl.ds(i * BI, BI), :]
    outd_ref[...] = jnp.maximum(
        jnp.dot(dual_ref[...], wd1_ref[...],
                preferred_element_type=jnp.float32)
        + jnp.dot(f_blk, wd2_ref[...], preferred_element_type=jnp.float32)
        + bd_ref[...], 0.0)


def _tc_fused(A, f, primal, dual, wp1, wp2, wd1, wd2, bp, bd):
    out_shape = (
        jax.ShapeDtypeStruct((N_V, DIM), jnp.float32),
        jax.ShapeDtypeStruct((N_F, DIM), jnp.float32),
    )
    dimdim = pl.BlockSpec((DIM, DIM), lambda i: (0, 0))
    rowblk = pl.BlockSpec((BI, DIM), lambda i: (i, 0))
    bias = pl.BlockSpec((1, DIM), lambda i: (0, 0))
    return pl.pallas_call(
        _tc_fused_body,
        grid=(NBI,),
        in_specs=[
            pl.BlockSpec((BI, N_F), lambda i: (i, 0)),
            pl.BlockSpec((N_F, DIM), lambda i: (0, 0)),
            rowblk,
            rowblk,
            dimdim, dimdim, dimdim, dimdim,
            bias, bias,
        ],
        out_specs=(rowblk, rowblk),
        out_shape=out_shape,
        compiler_params=pltpu.CompilerParams(
            dimension_semantics=("arbitrary",),
        ),
    )(A, f, primal, dual, wp1, wp2, wd1, wd2, bp, bd)


def kernel(primal, dual, A, faces, W_primal, b_primal, W_dual, b_dual):
    faces = faces.astype(jnp.int32)
    pad = IPAD - N_F
    i0 = jnp.pad(faces[:, 0], (0, pad))
    i1 = jnp.pad(faces[:, 1], (0, pad))
    i2 = jnp.pad(faces[:, 2], (0, pad))
    f = _face_features_sc(primal, dual, i0, i1, i2)
    wp1 = W_primal[:, :DIM].T
    wp2 = W_primal[:, DIM:].T
    wd1 = W_dual[:, :DIM].T
    wd2 = W_dual[:, DIM:].T
    bp = b_primal.reshape(1, DIM)
    bd = b_dual.reshape(1, DIM)
    out_primal, out_dual = _tc_fused(A, f, primal, dual,
                                     wp1, wp2, wd1, wd2, bp, bd)
    return (out_primal, out_dual)
